# baseline (speedup 1.0000x reference)
import functools

import jax
import jax.numpy as jnp
from jax import lax
from jax.experimental import pallas as pl
from jax.experimental.pallas import tpu as pltpu
from jax.experimental.pallas import tpu_sc as plsc

N = 10000
E = 320000
IN = 128
HID = 64
OUT = 2

NC = 2
NS = 16
NW = NC * NS
PER_W = E // NW
B1 = 400
G1 = B1 // 16
NCH1 = PER_W // B1
B2 = 2000
G2 = B2 // 16
NCH2 = PER_W // B2
ROWS_PER_CP = N // 10


def _l1_edges(xl_hbm, xr_hbm, src_hbm, dst_hbm, attv_hbm, zero80_hbm,
              numer_out,
              src_v, dst_v, rows_l, rows_r, stage, att_v,
              numer_s, sem1, sem2):
  c = lax.axis_index("c")
  s = lax.axis_index("s")
  wid = c * NS + s

  @pl.when(s < 10)
  def _():
    pltpu.sync_copy(zero80_hbm.at[pl.ds(s * ROWS_PER_CP, ROWS_PER_CP)],
                    numer_s.at[pl.ds(s * ROWS_PER_CP, ROWS_PER_CP)])

  pltpu.sync_copy(attv_hbm, att_v)
  plsc.subcore_barrier()

  lane = jnp.arange(16, dtype=jnp.int32)
  cidx = [jnp.full((16,), c, jnp.int32) for c in range(HID + 4)]

  def chunk(i, carry):
    base = wid * PER_W + i * B1
    pltpu.sync_copy(src_hbm.at[pl.ds(base, B1)], src_v)
    pltpu.sync_copy(dst_hbm.at[pl.ds(base, B1)], dst_v)
    cp1 = pltpu.async_copy(xl_hbm.at[src_v], rows_l, sem1)
    cp2 = pltpu.async_copy(xr_hbm.at[dst_v], rows_r, sem2)
    cp1.wait()
    cp2.wait()

    @plsc.parallel_loop(0, G1, 1, unroll=2)
    def _(g):
      e16 = g * 16 + lane
      accs = [jnp.zeros((16,), jnp.float32) for _ in range(4)]
      for c in range(HID):
        gl = plsc.load_gather(rows_l, [e16, cidx[c]])
        gr = plsc.load_gather(rows_r, [e16, cidx[c]])
        t = gl + gr
        t = jnp.maximum(t, t * 0.2)
        accs[c % 4] = accs[c % 4] + att_v[c, :] * t
      W = jnp.exp((accs[0] + accs[1]) + (accs[2] + accs[3]))
      for c in range(HID):
        gl = plsc.load_gather(rows_l, [e16, cidx[c]])
        plsc.store_scatter(stage, [e16, cidx[c]], gl * W)
      for c in range(HID, HID + 4):
        plsc.store_scatter(stage, [e16, cidx[c]], W)

    pltpu.sync_copy(stage, numer_s.at[dst_v], add=True)
    return carry

  lax.fori_loop(0, NCH1, chunk, 0)
  plsc.subcore_barrier()

  @pl.when(s < 10)
  def _():
    pltpu.sync_copy(numer_s.at[pl.ds(s * ROWS_PER_CP, ROWS_PER_CP)],
                    numer_out.at[pl.ds(c * N + s * ROWS_PER_CP, ROWS_PER_CP)])


def _l2_edges(tab_hbm, src_hbm, dst_hbm, att2v_hbm, zero4_hbm, acc_out,
              tab_v, src_v, dst_v, rows2, att2_v, acc_s, sem1):
  c = lax.axis_index("c")
  s = lax.axis_index("s")
  wid = c * NS + s

  @pl.when(s < 10)
  def _():
    pltpu.sync_copy(zero4_hbm.at[pl.ds(s * ROWS_PER_CP, ROWS_PER_CP)],
                    acc_s.at[pl.ds(s * ROWS_PER_CP, ROWS_PER_CP)])

  pltpu.sync_copy(tab_hbm, tab_v)
  pltpu.sync_copy(att2v_hbm, att2_v)
  plsc.subcore_barrier()

  lane = jnp.arange(16, dtype=jnp.int32)
  i0 = jnp.zeros((16,), jnp.int32)
  i1 = i0 + 1
  i2 = i0 + 2
  i3 = i0 + 3
  a0 = att2_v[0, :]
  a1 = att2_v[1, :]
  zf = jnp.zeros((16,), jnp.float32)

  def chunk(i, carry):
    base = wid * PER_W + i * B2
    pltpu.sync_copy(src_hbm.at[pl.ds(base, B2)], src_v)
    pltpu.sync_copy(dst_hbm.at[pl.ds(base, B2)], dst_v)

    def group(g, _):
      sv = src_v[pl.ds(g * 16, 16)]
      dv = dst_v[pl.ds(g * 16, 16)]
      l0 = plsc.load_gather(tab_v, [sv, i0])
      l1 = plsc.load_gather(tab_v, [sv, i1])
      r0 = plsc.load_gather(tab_v, [dv, i2])
      r1 = plsc.load_gather(tab_v, [dv, i3])
      t0 = l0 + r0
      t0 = jnp.maximum(t0, t0 * 0.2)
      t1 = l1 + r1
      t1 = jnp.maximum(t1, t1 * 0.2)
      w = jnp.exp(a0 * t0 + a1 * t1)
      eidx = g * 16 + lane
      plsc.store_scatter(rows2, [eidx, i0], w * l0)
      plsc.store_scatter(rows2, [eidx, i1], w * l1)
      plsc.store_scatter(rows2, [eidx, i2], w)
      plsc.store_scatter(rows2, [eidx, i3], zf)
      return 0

    lax.fori_loop(0, G2, group, 0)
    pltpu.sync_copy(rows2, acc_s.at[dst_v], add=True)
    return carry

  lax.fori_loop(0, NCH2, chunk, 0)
  plsc.subcore_barrier()

  @pl.when(s < 10)
  def _():
    pltpu.sync_copy(acc_s.at[pl.ds(s * ROWS_PER_CP, ROWS_PER_CP)],
                    acc_out.at[pl.ds(c * N + s * ROWS_PER_CP, ROWS_PER_CP)])


def _tc_in_body(x_ref, w_ref, b_ref, xl_ref, xr_ref):
  y = jnp.dot(x_ref[...], w_ref[...], preferred_element_type=jnp.float32)
  y = y + b_ref[...]
  xl_ref[...] = y[:, :HID]
  xr_ref[...] = y[:, HID:]


def _tc_mid_body(p0_ref, p1_ref, b1_ref, w2_ref, b2_ref, o_ref):
  p = p0_ref[...] + p1_ref[...]
  h = p[:, :HID] / (p[:, HID:HID + 1] + 1e-16)
  h = h + b1_ref[...]
  h = jnp.where(h > 0, h, jnp.exp(jnp.minimum(h, 0.0)) - 1.0)
  o_ref[...] = (
      jnp.dot(h, w2_ref[...], preferred_element_type=jnp.float32)
      + b2_ref[...]
  )


def _tc_fin_body(a0_ref, a1_ref, b_ref, o_ref):
  a = a0_ref[...] + a1_ref[...]
  o_ref[...] = a[:, :OUT] / (a[:, OUT:OUT + 1] + 1e-16) + b_ref[...]


_ROWBLK = 2000


def _tc_in(x, wcat_t, bcat):
  return pl.pallas_call(
      _tc_in_body,
      grid=(N // _ROWBLK,),
      in_specs=[
          pl.BlockSpec((_ROWBLK, IN), lambda i: (i, 0)),
          pl.BlockSpec((IN, 2 * HID), lambda i: (0, 0)),
          pl.BlockSpec((1, 2 * HID), lambda i: (0, 0)),
      ],
      out_specs=[
          pl.BlockSpec((_ROWBLK, HID), lambda i: (i, 0)),
          pl.BlockSpec((_ROWBLK, HID), lambda i: (i, 0)),
      ],
      out_shape=[
          jax.ShapeDtypeStruct((N, HID), jnp.float32),
          jax.ShapeDtypeStruct((N, HID), jnp.float32),
      ],
  )(x, wcat_t, bcat)


def _tc_mid(p0, p1, b1, w2t, b2):
  return pl.pallas_call(
      _tc_mid_body,
      grid=(N // _ROWBLK,),
      in_specs=[
          pl.BlockSpec((_ROWBLK, 68), lambda i: (i, 0)),
          pl.BlockSpec((_ROWBLK, 68), lambda i: (i, 0)),
          pl.BlockSpec((1, HID), lambda i: (0, 0)),
          pl.BlockSpec((HID, 4), lambda i: (0, 0)),
          pl.BlockSpec((1, 4), lambda i: (0, 0)),
      ],
      out_specs=pl.BlockSpec((_ROWBLK, 4), lambda i: (i, 0)),
      out_shape=jax.ShapeDtypeStruct((N, 4), jnp.float32),
  )(p0, p1, b1, w2t, b2)


def _tc_fin(a0, a1, b2):
  return pl.pallas_call(
      _tc_fin_body,
      grid=(N // _ROWBLK,),
      in_specs=[
          pl.BlockSpec((_ROWBLK, 4), lambda i: (i, 0)),
          pl.BlockSpec((_ROWBLK, 4), lambda i: (i, 0)),
          pl.BlockSpec((1, OUT), lambda i: (0, 0)),
      ],
      out_specs=pl.BlockSpec((_ROWBLK, OUT), lambda i: (i, 0)),
      out_shape=jax.ShapeDtypeStruct((N, OUT), jnp.float32),
  )(a0, a1, b2)


def kernel(x, edge_index, batch, Wl1, bl1, Wr1, br1, att1, bias1,
           Wl2, bl2, Wr2, br2, att2, bias2):
  del batch
  src = edge_index[0]
  dst = edge_index[1]

  wcat_t = jnp.concatenate([Wl1, Wr1], axis=0).T
  bcat = jnp.concatenate([bl1, br1], axis=0)[None, :]
  xl, xr = _tc_in(x, wcat_t, bcat)

  attv = jnp.broadcast_to(att1.reshape(HID, 1), (HID, 16))
  zero80 = jnp.zeros((N, 68), jnp.float32)

  mesh = plsc.VectorSubcoreMesh(core_axis_name="c", subcore_axis_name="s",
                                num_cores=NC, num_subcores=NS)
  sc_params = pltpu.CompilerParams(needs_layout_passes=False,
                                   use_tc_tiling_on_sc=False)
  l1 = pl.kernel(
      _l1_edges,
      compiler_params=sc_params,
      out_type=jax.ShapeDtypeStruct((NC * N, 68), jnp.float32),
      mesh=mesh,
      scratch_types=[
          pltpu.VMEM((B1,), jnp.int32),
          pltpu.VMEM((B1,), jnp.int32),
          pltpu.VMEM((B1, HID), jnp.float32),
          pltpu.VMEM((B1, HID), jnp.float32),
          pltpu.VMEM((B1, 68), jnp.float32),
          pltpu.VMEM((HID, 16), jnp.float32),
          pltpu.MemorySpace.VMEM_SHARED((N, 68), jnp.float32),
          pltpu.SemaphoreType.DMA,
          pltpu.SemaphoreType.DMA,
      ],
  )
  numer = l1(xl, xr, src, dst, attv, zero80)

  w2t = jnp.concatenate([Wl2, Wr2], axis=0).T
  b2 = jnp.concatenate([bl2, br2], axis=0)[None, :]
  tab2 = _tc_mid(numer[:N], numer[N:], bias1[None, :], w2t, b2)

  att2v = jnp.broadcast_to(att2.reshape(2, 1), (2, 16))
  zero4 = jnp.zeros((N, 4), jnp.float32)

  l2 = pl.kernel(
      _l2_edges,
      compiler_params=sc_params,
      out_type=jax.ShapeDtypeStruct((NC * N, 4), jnp.float32),
      mesh=mesh,
      scratch_types=[
          pltpu.VMEM((N, 4), jnp.float32),
          pltpu.VMEM((B2,), jnp.int32),
          pltpu.VMEM((B2,), jnp.int32),
          pltpu.VMEM((B2, 4), jnp.float32),
          pltpu.VMEM((2, 16), jnp.float32),
          pltpu.MemorySpace.VMEM_SHARED((N, 4), jnp.float32),
          pltpu.SemaphoreType.DMA,
      ],
  )
  acc = l2(tab2, src, dst, att2v, zero4)

  return _tc_fin(acc[:N], acc[N:], bias2[None, :])


# speedup vs baseline: 2.0717x; 2.0717x over previous
import functools

import jax
import jax.numpy as jnp
from jax import lax
from jax.experimental import pallas as pl
from jax.experimental.pallas import tpu as pltpu
from jax.experimental.pallas import tpu_sc as plsc

N = 10000
E = 320000
IN = 128
HID = 64
OUT = 2

NC = 2
NS = 16
NW = NC * NS
PER_W = E // NW
B1 = 400
G1 = B1 // 16
NCH1 = PER_W // B1
B2 = 2000
G2 = B2 // 16
NCH2 = PER_W // B2
ROWS_PER_CP = N // 10


def _l1_edges(xl_hbm, xr_hbm, src_hbm, dst_hbm, attv_hbm, zero80_hbm,
              numer_out,
              src_v, dst_v, rows_l, rows_r, stage, att_v, acc_m,
              numer_s, sem1, sem2):
  c = lax.axis_index("c")
  s = lax.axis_index("s")
  wid = c * NS + s

  @pl.when(s < 10)
  def _():
    pltpu.sync_copy(zero80_hbm.at[pl.ds(s * ROWS_PER_CP, ROWS_PER_CP)],
                    numer_s.at[pl.ds(s * ROWS_PER_CP, ROWS_PER_CP)])

  pltpu.sync_copy(attv_hbm, att_v)
  plsc.subcore_barrier()

  a_vecs = [att_v[k, :] for k in range(4)]
  lane = jnp.arange(16, dtype=jnp.int32)

  def chunk(i, carry):
    base = wid * PER_W + i * B1
    pltpu.sync_copy(src_hbm.at[pl.ds(base, B1)], src_v)
    pltpu.sync_copy(dst_hbm.at[pl.ds(base, B1)], dst_v)
    cp1 = pltpu.async_copy(xl_hbm.at[src_v], rows_l, sem1)
    cp2 = pltpu.async_copy(xr_hbm.at[dst_v], rows_r, sem2)
    cp1.wait()
    cp2.wait()

    def group(g, _):
      for j in range(16):
        e = g * 16 + j
        acc01 = jnp.zeros((16,), jnp.float32)
        acc23 = jnp.zeros((16,), jnp.float32)
        for k in range(4):
          vl = rows_l[e, pl.ds(k * 16, 16)]
          vr = rows_r[e, pl.ds(k * 16, 16)]
          t = vl + vr
          t = jnp.maximum(t, t * 0.2)
          if k % 2 == 0:
            acc01 = acc01 + a_vecs[k] * t
          else:
            acc23 = acc23 + a_vecs[k] * t
        acc_m[j, pl.ds(0, 16)] = acc01 + acc23
      S = jnp.zeros((16,), jnp.float32)
      for k in range(16):
        S = S + plsc.load_gather(acc_m, [lane, jnp.full((16,), k, jnp.int32)])
      W = jnp.exp(S)
      for j in range(16):
        e = g * 16 + j
        wv = jnp.broadcast_to(W[j], (16,))
        stage[e, pl.ds(52, 16)] = wv
        for k in range(4):
          stage[e, pl.ds(k * 16, 16)] = rows_l[e, pl.ds(k * 16, 16)] * wv
      return 0

    lax.fori_loop(0, G1, group, 0)
    pltpu.sync_copy(stage, numer_s.at[dst_v], add=True)
    return carry

  lax.fori_loop(0, NCH1, chunk, 0)
  plsc.subcore_barrier()

  @pl.when(s < 10)
  def _():
    pltpu.sync_copy(numer_s.at[pl.ds(s * ROWS_PER_CP, ROWS_PER_CP)],
                    numer_out.at[pl.ds(c * N + s * ROWS_PER_CP, ROWS_PER_CP)])


def _l2_edges(tab_hbm, src_hbm, dst_hbm, att2v_hbm, zero4_hbm, acc_out,
              tab_v, src_v, dst_v, rows2, att2_v, acc_s, sem1):
  c = lax.axis_index("c")
  s = lax.axis_index("s")
  wid = c * NS + s

  @pl.when(s < 10)
  def _():
    pltpu.sync_copy(zero4_hbm.at[pl.ds(s * ROWS_PER_CP, ROWS_PER_CP)],
                    acc_s.at[pl.ds(s * ROWS_PER_CP, ROWS_PER_CP)])

  pltpu.sync_copy(tab_hbm, tab_v)
  pltpu.sync_copy(att2v_hbm, att2_v)
  plsc.subcore_barrier()

  lane = jnp.arange(16, dtype=jnp.int32)
  i0 = jnp.zeros((16,), jnp.int32)
  i1 = i0 + 1
  i2 = i0 + 2
  i3 = i0 + 3
  a0 = att2_v[0, :]
  a1 = att2_v[1, :]
  zf = jnp.zeros((16,), jnp.float32)

  def chunk(i, carry):
    base = wid * PER_W + i * B2
    pltpu.sync_copy(src_hbm.at[pl.ds(base, B2)], src_v)
    pltpu.sync_copy(dst_hbm.at[pl.ds(base, B2)], dst_v)

    def group(g, _):
      sv = src_v[pl.ds(g * 16, 16)]
      dv = dst_v[pl.ds(g * 16, 16)]
      l0 = plsc.load_gather(tab_v, [sv, i0])
      l1 = plsc.load_gather(tab_v, [sv, i1])
      r0 = plsc.load_gather(tab_v, [dv, i2])
      r1 = plsc.load_gather(tab_v, [dv, i3])
      t0 = l0 + r0
      t0 = jnp.maximum(t0, t0 * 0.2)
      t1 = l1 + r1
      t1 = jnp.maximum(t1, t1 * 0.2)
      w = jnp.exp(a0 * t0 + a1 * t1)
      eidx = g * 16 + lane
      plsc.store_scatter(rows2, [eidx, i0], w * l0)
      plsc.store_scatter(rows2, [eidx, i1], w * l1)
      plsc.store_scatter(rows2, [eidx, i2], w)
      plsc.store_scatter(rows2, [eidx, i3], zf)
      return 0

    lax.fori_loop(0, G2, group, 0)
    pltpu.sync_copy(rows2, acc_s.at[dst_v], add=True)
    return carry

  lax.fori_loop(0, NCH2, chunk, 0)
  plsc.subcore_barrier()

  @pl.when(s < 10)
  def _():
    pltpu.sync_copy(acc_s.at[pl.ds(s * ROWS_PER_CP, ROWS_PER_CP)],
                    acc_out.at[pl.ds(c * N + s * ROWS_PER_CP, ROWS_PER_CP)])


def _tc_in_body(x_ref, w_ref, b_ref, xl_ref, xr_ref):
  y = jnp.dot(x_ref[...], w_ref[...], preferred_element_type=jnp.float32)
  y = y + b_ref[...]
  xl_ref[...] = y[:, :HID]
  xr_ref[...] = y[:, HID:]


def _tc_mid_body(p0_ref, p1_ref, b1_ref, w2_ref, b2_ref, o_ref):
  p = p0_ref[...] + p1_ref[...]
  h = p[:, :HID] / (p[:, HID:HID + 1] + 1e-16)
  h = h + b1_ref[...]
  h = jnp.where(h > 0, h, jnp.exp(jnp.minimum(h, 0.0)) - 1.0)
  o_ref[...] = (
      jnp.dot(h, w2_ref[...], preferred_element_type=jnp.float32)
      + b2_ref[...]
  )


def _tc_fin_body(a0_ref, a1_ref, b_ref, o_ref):
  a = a0_ref[...] + a1_ref[...]
  o_ref[...] = a[:, :OUT] / (a[:, OUT:OUT + 1] + 1e-16) + b_ref[...]


_ROWBLK = 2000


def _tc_in(x, wcat_t, bcat):
  return pl.pallas_call(
      _tc_in_body,
      grid=(N // _ROWBLK,),
      in_specs=[
          pl.BlockSpec((_ROWBLK, IN), lambda i: (i, 0)),
          pl.BlockSpec((IN, 2 * HID), lambda i: (0, 0)),
          pl.BlockSpec((1, 2 * HID), lambda i: (0, 0)),
      ],
      out_specs=[
          pl.BlockSpec((_ROWBLK, HID), lambda i: (i, 0)),
          pl.BlockSpec((_ROWBLK, HID), lambda i: (i, 0)),
      ],
      out_shape=[
          jax.ShapeDtypeStruct((N, HID), jnp.float32),
          jax.ShapeDtypeStruct((N, HID), jnp.float32),
      ],
  )(x, wcat_t, bcat)


def _tc_mid(p0, p1, b1, w2t, b2):
  return pl.pallas_call(
      _tc_mid_body,
      grid=(N // _ROWBLK,),
      in_specs=[
          pl.BlockSpec((_ROWBLK, 68), lambda i: (i, 0)),
          pl.BlockSpec((_ROWBLK, 68), lambda i: (i, 0)),
          pl.BlockSpec((1, HID), lambda i: (0, 0)),
          pl.BlockSpec((HID, 4), lambda i: (0, 0)),
          pl.BlockSpec((1, 4), lambda i: (0, 0)),
      ],
      out_specs=pl.BlockSpec((_ROWBLK, 4), lambda i: (i, 0)),
      out_shape=jax.ShapeDtypeStruct((N, 4), jnp.float32),
  )(p0, p1, b1, w2t, b2)


def _tc_fin(a0, a1, b2):
  return pl.pallas_call(
      _tc_fin_body,
      grid=(N // _ROWBLK,),
      in_specs=[
          pl.BlockSpec((_ROWBLK, 4), lambda i: (i, 0)),
          pl.BlockSpec((_ROWBLK, 4), lambda i: (i, 0)),
          pl.BlockSpec((1, OUT), lambda i: (0, 0)),
      ],
      out_specs=pl.BlockSpec((_ROWBLK, OUT), lambda i: (i, 0)),
      out_shape=jax.ShapeDtypeStruct((N, OUT), jnp.float32),
  )(a0, a1, b2)


def kernel(x, edge_index, batch, Wl1, bl1, Wr1, br1, att1, bias1,
           Wl2, bl2, Wr2, br2, att2, bias2):
  del batch
  src = edge_index[0]
  dst = edge_index[1]

  wcat_t = jnp.concatenate([Wl1, Wr1], axis=0).T
  bcat = jnp.concatenate([bl1, br1], axis=0)[None, :]
  xl, xr = _tc_in(x, wcat_t, bcat)

  attv = att1.reshape(4, 16)
  zero80 = jnp.zeros((N, 68), jnp.float32)

  mesh = plsc.VectorSubcoreMesh(core_axis_name="c", subcore_axis_name="s",
                                num_cores=NC, num_subcores=NS)
  sc_params = pltpu.CompilerParams(needs_layout_passes=False,
                                   use_tc_tiling_on_sc=False)
  l1 = pl.kernel(
      _l1_edges,
      compiler_params=sc_params,
      out_type=jax.ShapeDtypeStruct((NC * N, 68), jnp.float32),
      mesh=mesh,
      scratch_types=[
          pltpu.VMEM((B1,), jnp.int32),
          pltpu.VMEM((B1,), jnp.int32),
          pltpu.VMEM((B1, HID), jnp.float32),
          pltpu.VMEM((B1, HID), jnp.float32),
          pltpu.VMEM((B1, 68), jnp.float32),
          pltpu.VMEM((4, 16), jnp.float32),
          pltpu.VMEM((16, 17), jnp.float32),
          pltpu.MemorySpace.VMEM_SHARED((N, 68), jnp.float32),
          pltpu.SemaphoreType.DMA,
          pltpu.SemaphoreType.DMA,
      ],
  )
  numer = l1(xl, xr, src, dst, attv, zero80)

  w2t = jnp.concatenate([Wl2, Wr2], axis=0).T
  b2 = jnp.concatenate([bl2, br2], axis=0)[None, :]
  tab2 = _tc_mid(numer[:N], numer[N:], bias1[None, :], w2t, b2)

  att2v = jnp.broadcast_to(att2.reshape(2, 1), (2, 16))
  zero4 = jnp.zeros((N, 4), jnp.float32)

  l2 = pl.kernel(
      _l2_edges,
      compiler_params=sc_params,
      out_type=jax.ShapeDtypeStruct((NC * N, 4), jnp.float32),
      mesh=mesh,
      scratch_types=[
          pltpu.VMEM((N, 4), jnp.float32),
          pltpu.VMEM((B2,), jnp.int32),
          pltpu.VMEM((B2,), jnp.int32),
          pltpu.VMEM((B2, 4), jnp.float32),
          pltpu.VMEM((2, 16), jnp.float32),
          pltpu.MemorySpace.VMEM_SHARED((N, 4), jnp.float32),
          pltpu.SemaphoreType.DMA,
      ],
  )
  acc = l2(tab2, src, dst, att2v, zero4)

  return _tc_fin(acc[:N], acc[N:], bias2[None, :])


# speedup vs baseline: 2.1008x; 1.0141x over previous
import functools

import jax
import jax.numpy as jnp
from jax import lax
from jax.experimental import pallas as pl
from jax.experimental.pallas import tpu as pltpu
from jax.experimental.pallas import tpu_sc as plsc

N = 10000
E = 320000
IN = 128
HID = 64
OUT = 2

NC = 2
NS = 16
NW = NC * NS
PER_W = E // NW
B1 = 400
G1 = B1 // 16
NCH1 = PER_W // B1
B2 = 2000
G2 = B2 // 16
NCH2 = PER_W // B2
ROWS_PER_CP = N // 10


def _l1_edges(xl_hbm, xr_hbm, src_hbm, dst_hbm, attv_hbm, zero80_hbm,
              numer_out,
              src_v, dst_v, rows_l, rows_r, stage, att_v, acc_m,
              numer_s, sem1, sem2):
  c = lax.axis_index("c")
  s = lax.axis_index("s")
  wid = c * NS + s

  @pl.when(s < 10)
  def _():
    pltpu.sync_copy(zero80_hbm.at[pl.ds(s * ROWS_PER_CP, ROWS_PER_CP)],
                    numer_s.at[pl.ds(s * ROWS_PER_CP, ROWS_PER_CP)])

  pltpu.sync_copy(attv_hbm, att_v)
  plsc.subcore_barrier()

  a_vecs = [att_v[k, :] for k in range(4)]
  lane = jnp.arange(16, dtype=jnp.int32)

  def chunk(i, carry):
    base = wid * PER_W + i * B1
    pltpu.sync_copy(src_hbm.at[pl.ds(base, B1)], src_v)
    pltpu.sync_copy(dst_hbm.at[pl.ds(base, B1)], dst_v)
    cp1 = pltpu.async_copy(xl_hbm.at[src_v], rows_l, sem1)
    cp2 = pltpu.async_copy(xr_hbm.at[dst_v], rows_r, sem2)
    cp1.wait()
    cp2.wait()

    def pass1(g, p):
      for j in range(16):
        e = g * 16 + j
        acc01 = jnp.zeros((16,), jnp.float32)
        acc23 = jnp.zeros((16,), jnp.float32)
        for k in range(4):
          vl = rows_l[e, pl.ds(k * 16, 16)]
          vr = rows_r[e, pl.ds(k * 16, 16)]
          t = vl + vr
          t = jnp.maximum(t, t * 0.2)
          if k % 2 == 0:
            acc01 = acc01 + a_vecs[k] * t
          else:
            acc23 = acc23 + a_vecs[k] * t
        acc_m[p * 16 + j, pl.ds(0, 16)] = acc01 + acc23

    def pass2(g, p):
      S = jnp.zeros((16,), jnp.float32)
      for k in range(16):
        S = S + plsc.load_gather(
            acc_m, [p * 16 + lane, jnp.full((16,), k, jnp.int32)])
      W = jnp.exp(S)
      for j in range(16):
        e = g * 16 + j
        wv = jnp.broadcast_to(W[j], (16,))
        stage[e, pl.ds(52, 16)] = wv
        for k in range(4):
          stage[e, pl.ds(k * 16, 16)] = rows_l[e, pl.ds(k * 16, 16)] * wv

    def group2(h, _):
      pass1(h * 2, 0)
      pass1(h * 2 + 1, 1)
      pass2(h * 2, 0)
      pass2(h * 2 + 1, 1)
      return 0

    lax.fori_loop(0, G1 // 2, group2, 0)
    pass1(G1 - 1, 0)
    pass2(G1 - 1, 0)
    pltpu.sync_copy(stage, numer_s.at[dst_v], add=True)
    return carry

  lax.fori_loop(0, NCH1, chunk, 0)
  plsc.subcore_barrier()

  @pl.when(s < 10)
  def _():
    pltpu.sync_copy(numer_s.at[pl.ds(s * ROWS_PER_CP, ROWS_PER_CP)],
                    numer_out.at[pl.ds(c * N + s * ROWS_PER_CP, ROWS_PER_CP)])


def _l2_edges(tab_hbm, src_hbm, dst_hbm, att2v_hbm, zero4_hbm, acc_out,
              tab_v, src_v, dst_v, rows2, att2_v, acc_s, sem1):
  c = lax.axis_index("c")
  s = lax.axis_index("s")
  wid = c * NS + s

  @pl.when(s < 10)
  def _():
    pltpu.sync_copy(zero4_hbm.at[pl.ds(s * ROWS_PER_CP, ROWS_PER_CP)],
                    acc_s.at[pl.ds(s * ROWS_PER_CP, ROWS_PER_CP)])

  pltpu.sync_copy(tab_hbm, tab_v)
  pltpu.sync_copy(att2v_hbm, att2_v)
  plsc.subcore_barrier()

  lane = jnp.arange(16, dtype=jnp.int32)
  i0 = jnp.zeros((16,), jnp.int32)
  i1 = i0 + 1
  i2 = i0 + 2
  i3 = i0 + 3
  a0 = att2_v[0, :]
  a1 = att2_v[1, :]
  zf = jnp.zeros((16,), jnp.float32)

  def chunk(i, carry):
    base = wid * PER_W + i * B2
    pltpu.sync_copy(src_hbm.at[pl.ds(base, B2)], src_v)
    pltpu.sync_copy(dst_hbm.at[pl.ds(base, B2)], dst_v)

    def group(g, _):
      sv = src_v[pl.ds(g * 16, 16)]
      dv = dst_v[pl.ds(g * 16, 16)]
      l0 = plsc.load_gather(tab_v, [sv, i0])
      l1 = plsc.load_gather(tab_v, [sv, i1])
      r0 = plsc.load_gather(tab_v, [dv, i2])
      r1 = plsc.load_gather(tab_v, [dv, i3])
      t0 = l0 + r0
      t0 = jnp.maximum(t0, t0 * 0.2)
      t1 = l1 + r1
      t1 = jnp.maximum(t1, t1 * 0.2)
      w = jnp.exp(a0 * t0 + a1 * t1)
      eidx = g * 16 + lane
      plsc.store_scatter(rows2, [eidx, i0], w * l0)
      plsc.store_scatter(rows2, [eidx, i1], w * l1)
      plsc.store_scatter(rows2, [eidx, i2], w)
      plsc.store_scatter(rows2, [eidx, i3], zf)
      return 0

    lax.fori_loop(0, G2, group, 0)
    pltpu.sync_copy(rows2, acc_s.at[dst_v], add=True)
    return carry

  lax.fori_loop(0, NCH2, chunk, 0)
  plsc.subcore_barrier()

  @pl.when(s < 10)
  def _():
    pltpu.sync_copy(acc_s.at[pl.ds(s * ROWS_PER_CP, ROWS_PER_CP)],
                    acc_out.at[pl.ds(c * N + s * ROWS_PER_CP, ROWS_PER_CP)])


def _tc_in_body(x_ref, w_ref, b_ref, xl_ref, xr_ref):
  y = jnp.dot(x_ref[...], w_ref[...], preferred_element_type=jnp.float32)
  y = y + b_ref[...]
  xl_ref[...] = y[:, :HID]
  xr_ref[...] = y[:, HID:]


def _tc_mid_body(p0_ref, p1_ref, b1_ref, w2_ref, b2_ref, o_ref):
  p = p0_ref[...] + p1_ref[...]
  h = p[:, :HID] / (p[:, HID:HID + 1] + 1e-16)
  h = h + b1_ref[...]
  h = jnp.where(h > 0, h, jnp.exp(jnp.minimum(h, 0.0)) - 1.0)
  o_ref[...] = (
      jnp.dot(h, w2_ref[...], preferred_element_type=jnp.float32)
      + b2_ref[...]
  )


def _tc_fin_body(a0_ref, a1_ref, b_ref, o_ref):
  a = a0_ref[...] + a1_ref[...]
  o_ref[...] = a[:, :OUT] / (a[:, OUT:OUT + 1] + 1e-16) + b_ref[...]


_ROWBLK = 2000


def _tc_in(x, wcat_t, bcat):
  return pl.pallas_call(
      _tc_in_body,
      grid=(N // _ROWBLK,),
      in_specs=[
          pl.BlockSpec((_ROWBLK, IN), lambda i: (i, 0)),
          pl.BlockSpec((IN, 2 * HID), lambda i: (0, 0)),
          pl.BlockSpec((1, 2 * HID), lambda i: (0, 0)),
      ],
      out_specs=[
          pl.BlockSpec((_ROWBLK, HID), lambda i: (i, 0)),
          pl.BlockSpec((_ROWBLK, HID), lambda i: (i, 0)),
      ],
      out_shape=[
          jax.ShapeDtypeStruct((N, HID), jnp.float32),
          jax.ShapeDtypeStruct((N, HID), jnp.float32),
      ],
  )(x, wcat_t, bcat)


def _tc_mid(p0, p1, b1, w2t, b2):
  return pl.pallas_call(
      _tc_mid_body,
      grid=(N // _ROWBLK,),
      in_specs=[
          pl.BlockSpec((_ROWBLK, 68), lambda i: (i, 0)),
          pl.BlockSpec((_ROWBLK, 68), lambda i: (i, 0)),
          pl.BlockSpec((1, HID), lambda i: (0, 0)),
          pl.BlockSpec((HID, 4), lambda i: (0, 0)),
          pl.BlockSpec((1, 4), lambda i: (0, 0)),
      ],
      out_specs=pl.BlockSpec((_ROWBLK, 4), lambda i: (i, 0)),
      out_shape=jax.ShapeDtypeStruct((N, 4), jnp.float32),
  )(p0, p1, b1, w2t, b2)


def _tc_fin(a0, a1, b2):
  return pl.pallas_call(
      _tc_fin_body,
      grid=(N // _ROWBLK,),
      in_specs=[
          pl.BlockSpec((_ROWBLK, 4), lambda i: (i, 0)),
          pl.BlockSpec((_ROWBLK, 4), lambda i: (i, 0)),
          pl.BlockSpec((1, OUT), lambda i: (0, 0)),
      ],
      out_specs=pl.BlockSpec((_ROWBLK, OUT), lambda i: (i, 0)),
      out_shape=jax.ShapeDtypeStruct((N, OUT), jnp.float32),
  )(a0, a1, b2)


def kernel(x, edge_index, batch, Wl1, bl1, Wr1, br1, att1, bias1,
           Wl2, bl2, Wr2, br2, att2, bias2):
  del batch
  src = edge_index[0]
  dst = edge_index[1]

  wcat_t = jnp.concatenate([Wl1, Wr1], axis=0).T
  bcat = jnp.concatenate([bl1, br1], axis=0)[None, :]
  xl, xr = _tc_in(x, wcat_t, bcat)

  attv = att1.reshape(4, 16)
  zero80 = jnp.zeros((N, 68), jnp.float32)

  mesh = plsc.VectorSubcoreMesh(core_axis_name="c", subcore_axis_name="s",
                                num_cores=NC, num_subcores=NS)
  sc_params = pltpu.CompilerParams(needs_layout_passes=False,
                                   use_tc_tiling_on_sc=False)
  l1 = pl.kernel(
      _l1_edges,
      compiler_params=sc_params,
      out_type=jax.ShapeDtypeStruct((NC * N, 68), jnp.float32),
      mesh=mesh,
      scratch_types=[
          pltpu.VMEM((B1,), jnp.int32),
          pltpu.VMEM((B1,), jnp.int32),
          pltpu.VMEM((B1, HID), jnp.float32),
          pltpu.VMEM((B1, HID), jnp.float32),
          pltpu.VMEM((B1, 68), jnp.float32),
          pltpu.VMEM((4, 16), jnp.float32),
          pltpu.VMEM((32, 17), jnp.float32),
          pltpu.MemorySpace.VMEM_SHARED((N, 68), jnp.float32),
          pltpu.SemaphoreType.DMA,
          pltpu.SemaphoreType.DMA,
      ],
  )
  numer = l1(xl, xr, src, dst, attv, zero80)

  w2t = jnp.concatenate([Wl2, Wr2], axis=0).T
  b2 = jnp.concatenate([bl2, br2], axis=0)[None, :]
  tab2 = _tc_mid(numer[:N], numer[N:], bias1[None, :], w2t, b2)

  att2v = jnp.broadcast_to(att2.reshape(2, 1), (2, 16))
  zero4 = jnp.zeros((N, 4), jnp.float32)

  l2 = pl.kernel(
      _l2_edges,
      compiler_params=sc_params,
      out_type=jax.ShapeDtypeStruct((NC * N, 4), jnp.float32),
      mesh=mesh,
      scratch_types=[
          pltpu.VMEM((N, 4), jnp.float32),
          pltpu.VMEM((B2,), jnp.int32),
          pltpu.VMEM((B2,), jnp.int32),
          pltpu.VMEM((B2, 4), jnp.float32),
          pltpu.VMEM((2, 16), jnp.float32),
          pltpu.MemorySpace.VMEM_SHARED((N, 4), jnp.float32),
          pltpu.SemaphoreType.DMA,
      ],
  )
  acc = l2(tab2, src, dst, att2v, zero4)

  return _tc_fin(acc[:N], acc[N:], bias2[None, :])


# speedup vs baseline: 2.2592x; 1.0754x over previous
import functools

import jax
import jax.numpy as jnp
from jax import lax
from jax.experimental import pallas as pl
from jax.experimental.pallas import tpu as pltpu
from jax.experimental.pallas import tpu_sc as plsc

N = 10000
E = 320000
IN = 128
HID = 64
OUT = 2

NC = 2
NS = 16
NW = NC * NS
PER_W = E // NW
B1H = 192
GH = B1H // 16
NHALF = 52
NPAIR = NHALF // 2
L1_MAIN = NHALF * B1H
L1_TAIL = PER_W - L1_MAIN
B2 = 2000
G2 = B2 // 16
NCH2 = PER_W // B2
ROWS_PER_CP = N // 10


def _l1_edges(xl_hbm, xr_hbm, src_hbm, dst_hbm, srct_hbm, dstt_hbm,
              attv_hbm, zero80_hbm, numer_out,
              src_v0, src_v1, dst_v0, dst_v1, dst_sc0, dst_sc1,
              src_t, dst_t,
              rows_l0, rows_l1, rows_r0, rows_r1, stage0, stage1,
              att_v, acc_m, numer_s,
              sg0, sg1, ss0, ss1):
  c = lax.axis_index("c")
  s = lax.axis_index("s")
  wid = c * NS + s

  @pl.when(s < 10)
  def _():
    pltpu.sync_copy(zero80_hbm.at[pl.ds(s * ROWS_PER_CP, ROWS_PER_CP)],
                    numer_s.at[pl.ds(s * ROWS_PER_CP, ROWS_PER_CP)])

  pltpu.sync_copy(attv_hbm, att_v)
  pltpu.sync_copy(srct_hbm.at[wid], src_t)
  pltpu.sync_copy(dstt_hbm.at[wid], dst_t)
  plsc.subcore_barrier()
  ebase = wid * PER_W

  a_vecs = [att_v[k, :] for k in range(4)]
  lane = jnp.arange(16, dtype=jnp.int32)
  rows_l = (rows_l0, rows_l1)
  rows_r = (rows_r0, rows_r1)
  stage = (stage0, stage1)
  src_vs = (src_v0, src_v1)
  dst_vs = (dst_v0, dst_v1)
  dst_scs = (dst_sc0, dst_sc1)
  sg = (sg0, sg1)
  ss = (ss0, ss1)

  def issue_gather(h, p):
    pltpu.sync_copy(src_hbm.at[pl.ds(ebase + h * B1H, B1H)], src_vs[p])
    pltpu.sync_copy(dst_hbm.at[pl.ds(ebase + h * B1H, B1H)], dst_vs[p])
    pltpu.async_copy(xl_hbm.at[src_vs[p]], rows_l[p], sg[p])
    pltpu.async_copy(xr_hbm.at[dst_vs[p]], rows_r[p], sg[p])

  def wait_gather(h, p):
    del h
    pltpu.make_async_copy(xl_hbm.at[src_vs[p]], rows_l[p], sg[p]).wait()
    pltpu.make_async_copy(xr_hbm.at[dst_vs[p]], rows_r[p], sg[p]).wait()

  def issue_scatter(p):
    for q in range(B1H // 16):
      dst_scs[p][pl.ds(q * 16, 16)] = dst_vs[p][pl.ds(q * 16, 16)]
    pltpu.async_copy(stage[p], numer_s.at[dst_scs[p]], ss[p], add=True)

  def wait_scatter(p):
    pltpu.make_async_copy(stage[p], numer_s.at[dst_scs[p]], ss[p]).wait()

  def pass1(g, p, rl, rr):
    for j in range(16):
      e = g * 16 + j
      acc01 = jnp.zeros((16,), jnp.float32)
      acc23 = jnp.zeros((16,), jnp.float32)
      for k in range(4):
        vl = rl[e, pl.ds(k * 16, 16)]
        vr = rr[e, pl.ds(k * 16, 16)]
        t = vl + vr
        t = jnp.maximum(t, t * 0.2)
        if k % 2 == 0:
          acc01 = acc01 + a_vecs[k] * t
        else:
          acc23 = acc23 + a_vecs[k] * t
      acc_m[p * 16 + j, pl.ds(0, 16)] = acc01 + acc23

  def pass2(g, p, rl, st):
    S = jnp.zeros((16,), jnp.float32)
    for k in range(16):
      S = S + plsc.load_gather(
          acc_m, [p * 16 + lane, jnp.full((16,), k, jnp.int32)])
    W = jnp.exp(S)
    for j in range(16):
      e = g * 16 + j
      wv = jnp.broadcast_to(W[j], (16,))
      st[e, pl.ds(52, 16)] = wv
      for k in range(4):
        st[e, pl.ds(k * 16, 16)] = rl[e, pl.ds(k * 16, 16)] * wv

  def compute_half(p):
    rl, rr, st = rows_l[p], rows_r[p], stage[p]

    def group2(h2, _):
      pass1(h2 * 2, 0, rl, rr)
      pass1(h2 * 2 + 1, 1, rl, rr)
      pass2(h2 * 2, 0, rl, st)
      pass2(h2 * 2 + 1, 1, rl, st)
      return 0

    lax.fori_loop(0, GH // 2, group2, 0)

  issue_gather(0, 0)

  def pipeline(i2, carry):
    issue_gather(i2 * 2 + 1, 1)
    wait_gather(0, 0)

    @pl.when(i2 >= 1)
    def _():
      wait_scatter(0)

    compute_half(0)
    issue_scatter(0)

    @pl.when(i2 < NPAIR - 1)
    def _():
      issue_gather(i2 * 2 + 2, 0)

    wait_gather(0, 1)

    @pl.when(i2 >= 1)
    def _():
      wait_scatter(1)

    compute_half(1)
    issue_scatter(1)
    return carry

  lax.fori_loop(0, NPAIR, pipeline, 0)
  wait_scatter(0)
  wait_scatter(1)

  cp1 = pltpu.async_copy(xl_hbm.at[src_t], rows_l0.at[pl.ds(0, 16)], sg0)
  cp2 = pltpu.async_copy(xr_hbm.at[dst_t], rows_r0.at[pl.ds(0, 16)], sg1)
  cp1.wait()
  cp2.wait()
  pass1(0, 0, rows_l0, rows_r0)
  pass2(0, 0, rows_l0, stage0)
  pltpu.sync_copy(stage0.at[pl.ds(0, 16)], numer_s.at[dst_t], add=True)

  plsc.subcore_barrier()

  @pl.when(s < 10)
  def _():
    pltpu.sync_copy(numer_s.at[pl.ds(s * ROWS_PER_CP, ROWS_PER_CP)],
                    numer_out.at[pl.ds(c * N + s * ROWS_PER_CP, ROWS_PER_CP)])


def _l2_edges(tab_hbm, src_hbm, dst_hbm, att2v_hbm, zero4_hbm, acc_out,
              tab_v, src_v, dst_v, rows2, att2_v, acc_s, sem1):
  c = lax.axis_index("c")
  s = lax.axis_index("s")
  wid = c * NS + s

  @pl.when(s < 10)
  def _():
    pltpu.sync_copy(zero4_hbm.at[pl.ds(s * ROWS_PER_CP, ROWS_PER_CP)],
                    acc_s.at[pl.ds(s * ROWS_PER_CP, ROWS_PER_CP)])

  pltpu.sync_copy(tab_hbm, tab_v)
  pltpu.sync_copy(att2v_hbm, att2_v)
  plsc.subcore_barrier()

  lane = jnp.arange(16, dtype=jnp.int32)
  i0 = jnp.zeros((16,), jnp.int32)
  i1 = i0 + 1
  i2 = i0 + 2
  i3 = i0 + 3
  a0 = att2_v[0, :]
  a1 = att2_v[1, :]
  zf = jnp.zeros((16,), jnp.float32)

  def chunk(i, carry):
    base = wid * PER_W + i * B2
    pltpu.sync_copy(src_hbm.at[pl.ds(base, B2)], src_v)
    pltpu.sync_copy(dst_hbm.at[pl.ds(base, B2)], dst_v)

    def group(g, _):
      sv = src_v[pl.ds(g * 16, 16)]
      dv = dst_v[pl.ds(g * 16, 16)]
      l0 = plsc.load_gather(tab_v, [sv, i0])
      l1 = plsc.load_gather(tab_v, [sv, i1])
      r0 = plsc.load_gather(tab_v, [dv, i2])
      r1 = plsc.load_gather(tab_v, [dv, i3])
      t0 = l0 + r0
      t0 = jnp.maximum(t0, t0 * 0.2)
      t1 = l1 + r1
      t1 = jnp.maximum(t1, t1 * 0.2)
      w = jnp.exp(a0 * t0 + a1 * t1)
      eidx = g * 16 + lane
      plsc.store_scatter(rows2, [eidx, i0], w * l0)
      plsc.store_scatter(rows2, [eidx, i1], w * l1)
      plsc.store_scatter(rows2, [eidx, i2], w)
      plsc.store_scatter(rows2, [eidx, i3], zf)
      return 0

    lax.fori_loop(0, G2, group, 0)
    pltpu.sync_copy(rows2, acc_s.at[dst_v], add=True)
    return carry

  lax.fori_loop(0, NCH2, chunk, 0)
  plsc.subcore_barrier()

  @pl.when(s < 10)
  def _():
    pltpu.sync_copy(acc_s.at[pl.ds(s * ROWS_PER_CP, ROWS_PER_CP)],
                    acc_out.at[pl.ds(c * N + s * ROWS_PER_CP, ROWS_PER_CP)])


def _tc_in_body(x_ref, w_ref, b_ref, xl_ref, xr_ref):
  y = jnp.dot(x_ref[...], w_ref[...], preferred_element_type=jnp.float32)
  y = y + b_ref[...]
  xl_ref[...] = y[:, :HID]
  xr_ref[...] = y[:, HID:]


def _tc_mid_body(p0_ref, p1_ref, b1_ref, w2_ref, b2_ref, o_ref):
  p = p0_ref[...] + p1_ref[...]
  h = p[:, :HID] / (p[:, HID:HID + 1] + 1e-16)
  h = h + b1_ref[...]
  h = jnp.where(h > 0, h, jnp.exp(jnp.minimum(h, 0.0)) - 1.0)
  o_ref[...] = (
      jnp.dot(h, w2_ref[...], preferred_element_type=jnp.float32)
      + b2_ref[...]
  )


def _tc_fin_body(a0_ref, a1_ref, b_ref, o_ref):
  a = a0_ref[...] + a1_ref[...]
  o_ref[...] = a[:, :OUT] / (a[:, OUT:OUT + 1] + 1e-16) + b_ref[...]


_ROWBLK = 2000


def _tc_in(x, wcat_t, bcat):
  return pl.pallas_call(
      _tc_in_body,
      grid=(N // _ROWBLK,),
      in_specs=[
          pl.BlockSpec((_ROWBLK, IN), lambda i: (i, 0)),
          pl.BlockSpec((IN, 2 * HID), lambda i: (0, 0)),
          pl.BlockSpec((1, 2 * HID), lambda i: (0, 0)),
      ],
      out_specs=[
          pl.BlockSpec((_ROWBLK, HID), lambda i: (i, 0)),
          pl.BlockSpec((_ROWBLK, HID), lambda i: (i, 0)),
      ],
      out_shape=[
          jax.ShapeDtypeStruct((N, HID), jnp.float32),
          jax.ShapeDtypeStruct((N, HID), jnp.float32),
      ],
  )(x, wcat_t, bcat)


def _tc_mid(p0, p1, b1, w2t, b2):
  return pl.pallas_call(
      _tc_mid_body,
      grid=(N // _ROWBLK,),
      in_specs=[
          pl.BlockSpec((_ROWBLK, 68), lambda i: (i, 0)),
          pl.BlockSpec((_ROWBLK, 68), lambda i: (i, 0)),
          pl.BlockSpec((1, HID), lambda i: (0, 0)),
          pl.BlockSpec((HID, 4), lambda i: (0, 0)),
          pl.BlockSpec((1, 4), lambda i: (0, 0)),
      ],
      out_specs=pl.BlockSpec((_ROWBLK, 4), lambda i: (i, 0)),
      out_shape=jax.ShapeDtypeStruct((N, 4), jnp.float32),
  )(p0, p1, b1, w2t, b2)


def _tc_fin(a0, a1, b2):
  return pl.pallas_call(
      _tc_fin_body,
      grid=(N // _ROWBLK,),
      in_specs=[
          pl.BlockSpec((_ROWBLK, 4), lambda i: (i, 0)),
          pl.BlockSpec((_ROWBLK, 4), lambda i: (i, 0)),
          pl.BlockSpec((1, OUT), lambda i: (0, 0)),
      ],
      out_specs=pl.BlockSpec((_ROWBLK, OUT), lambda i: (i, 0)),
      out_shape=jax.ShapeDtypeStruct((N, OUT), jnp.float32),
  )(a0, a1, b2)


def kernel(x, edge_index, batch, Wl1, bl1, Wr1, br1, att1, bias1,
           Wl2, bl2, Wr2, br2, att2, bias2):
  del batch
  src = edge_index[0]
  dst = edge_index[1]

  wcat_t = jnp.concatenate([Wl1, Wr1], axis=0).T
  bcat = jnp.concatenate([bl1, br1], axis=0)[None, :]
  xl, xr = _tc_in(x, wcat_t, bcat)

  attv = att1.reshape(4, 16)
  zero80 = jnp.zeros((N, 68), jnp.float32)

  srcr = src.reshape(NW, PER_W)
  dstr = dst.reshape(NW, PER_W)
  srct = srcr[:, L1_MAIN:]
  dstt = dstr[:, L1_MAIN:]

  mesh = plsc.VectorSubcoreMesh(core_axis_name="c", subcore_axis_name="s",
                                num_cores=NC, num_subcores=NS)
  sc_params = pltpu.CompilerParams(needs_layout_passes=False,
                                   use_tc_tiling_on_sc=False)
  l1 = pl.kernel(
      _l1_edges,
      compiler_params=sc_params,
      out_type=jax.ShapeDtypeStruct((NC * N, 68), jnp.float32),
      mesh=mesh,
      scratch_types=[
          pltpu.VMEM((B1H,), jnp.int32),
          pltpu.VMEM((B1H,), jnp.int32),
          pltpu.VMEM((B1H,), jnp.int32),
          pltpu.VMEM((B1H,), jnp.int32),
          pltpu.VMEM((B1H,), jnp.int32),
          pltpu.VMEM((B1H,), jnp.int32),
          pltpu.VMEM((L1_TAIL,), jnp.int32),
          pltpu.VMEM((L1_TAIL,), jnp.int32),
          pltpu.VMEM((B1H, HID), jnp.float32),
          pltpu.VMEM((B1H, HID), jnp.float32),
          pltpu.VMEM((B1H, HID), jnp.float32),
          pltpu.VMEM((B1H, HID), jnp.float32),
          pltpu.VMEM((B1H, 68), jnp.float32),
          pltpu.VMEM((B1H, 68), jnp.float32),
          pltpu.VMEM((4, 16), jnp.float32),
          pltpu.VMEM((32, 17), jnp.float32),
          pltpu.MemorySpace.VMEM_SHARED((N, 68), jnp.float32),
          pltpu.SemaphoreType.DMA,
          pltpu.SemaphoreType.DMA,
          pltpu.SemaphoreType.DMA,
          pltpu.SemaphoreType.DMA,
      ],
  )
  numer = l1(xl, xr, src, dst, srct, dstt, attv, zero80)

  w2t = jnp.concatenate([Wl2, Wr2], axis=0).T
  b2 = jnp.concatenate([bl2, br2], axis=0)[None, :]
  tab2 = _tc_mid(numer[:N], numer[N:], bias1[None, :], w2t, b2)

  att2v = jnp.broadcast_to(att2.reshape(2, 1), (2, 16))
  zero4 = jnp.zeros((N, 4), jnp.float32)

  l2 = pl.kernel(
      _l2_edges,
      compiler_params=sc_params,
      out_type=jax.ShapeDtypeStruct((NC * N, 4), jnp.float32),
      mesh=mesh,
      scratch_types=[
          pltpu.VMEM((N, 4), jnp.float32),
          pltpu.VMEM((B2,), jnp.int32),
          pltpu.VMEM((B2,), jnp.int32),
          pltpu.VMEM((B2, 4), jnp.float32),
          pltpu.VMEM((2, 16), jnp.float32),
          pltpu.MemorySpace.VMEM_SHARED((N, 4), jnp.float32),
          pltpu.SemaphoreType.DMA,
      ],
  )
  acc = l2(tab2, src, dst, att2v, zero4)

  return _tc_fin(acc[:N], acc[N:], bias2[None, :])


# speedup vs baseline: 2.4564x; 1.0873x over previous
import functools

import jax
import jax.numpy as jnp
from jax import lax
from jax.experimental import pallas as pl
from jax.experimental.pallas import tpu as pltpu
from jax.experimental.pallas import tpu_sc as plsc

N = 10000
E = 320000
IN = 128
HID = 64
OUT = 2

NC = 2
NS = 16
NW = NC * NS
PER_W = E // NW
B1H = 192
GH = B1H // 16
NHALF = 52
NPAIR = NHALF // 2
L1_MAIN = NHALF * B1H
L1_TAIL = PER_W - L1_MAIN
B2 = 2000
G2 = B2 // 16
NCH2 = PER_W // B2
ROWS_PER_CP = N // 10


def _l1_edges(xl_hbm, xr_hbm, src_hbm, dst_hbm, srct_hbm, dstt_hbm,
              attv_hbm, zero80_hbm, numer_out,
              src_v0, src_v1, dst_v0, dst_v1, dst_sc0, dst_sc1,
              src_t, dst_t,
              rows_l0, rows_l1, rows_r0, rows_r1, stage0, stage1,
              att_v, acc_m, numer_s,
              si0, si1, sg0, sg1, ss0, ss1):
  c = lax.axis_index("c")
  s = lax.axis_index("s")
  wid = c * NS + s

  @pl.when(s < 10)
  def _():
    pltpu.sync_copy(zero80_hbm.at[pl.ds(s * ROWS_PER_CP, ROWS_PER_CP)],
                    numer_s.at[pl.ds(s * ROWS_PER_CP, ROWS_PER_CP)])

  pltpu.sync_copy(attv_hbm, att_v)
  pltpu.sync_copy(srct_hbm.at[wid], src_t)
  pltpu.sync_copy(dstt_hbm.at[wid], dst_t)
  ebase = wid * PER_W
  plsc.subcore_barrier()

  a_vecs = [att_v[k, :] for k in range(4)]
  lane = jnp.arange(16, dtype=jnp.int32)
  rows_l = (rows_l0, rows_l1)
  rows_r = (rows_r0, rows_r1)
  stage = (stage0, stage1)
  src_vs = (src_v0, src_v1)
  dst_vs = (dst_v0, dst_v1)
  dst_scs = (dst_sc0, dst_sc1)
  si = (si0, si1)
  sg = (sg0, sg1)
  ss = (ss0, ss1)

  def issue_idx(h, p):
    pltpu.async_copy(src_hbm.at[pl.ds(ebase + h * B1H, B1H)], src_vs[p],
                     si[p])
    pltpu.async_copy(dst_hbm.at[pl.ds(ebase + h * B1H, B1H)], dst_vs[p],
                     si[p])

  def wait_idx(p):
    pltpu.make_async_copy(src_hbm.at[pl.ds(ebase, B1H)], src_vs[p],
                          si[p]).wait()
    pltpu.make_async_copy(dst_hbm.at[pl.ds(ebase, B1H)], dst_vs[p],
                          si[p]).wait()

  def issue_gather(p):
    pltpu.async_copy(xl_hbm.at[src_vs[p]], rows_l[p], sg[p])
    pltpu.async_copy(xr_hbm.at[dst_vs[p]], rows_r[p], sg[p])

  def wait_gather(p):
    pltpu.make_async_copy(xl_hbm.at[src_vs[p]], rows_l[p], sg[p]).wait()
    pltpu.make_async_copy(xr_hbm.at[dst_vs[p]], rows_r[p], sg[p]).wait()

  def issue_scatter(p):
    pltpu.async_copy(stage[p], numer_s.at[dst_scs[p]], ss[p], add=True)

  def wait_scatter(p):
    pltpu.make_async_copy(stage[p], numer_s.at[dst_scs[p]], ss[p]).wait()

  def pass1(g, p, rl, rr):
    for j in range(16):
      e = g * 16 + j
      acc01 = jnp.zeros((16,), jnp.float32)
      acc23 = jnp.zeros((16,), jnp.float32)
      for k in range(4):
        vl = rl[e, pl.ds(k * 16, 16)]
        vr = rr[e, pl.ds(k * 16, 16)]
        t = vl + vr
        t = jnp.maximum(t, t * 0.2)
        if k % 2 == 0:
          acc01 = acc01 + a_vecs[k] * t
        else:
          acc23 = acc23 + a_vecs[k] * t
      acc_m[p * 16 + j, pl.ds(0, 16)] = acc01 + acc23

  def pass2(g, p, rl, st):
    S = jnp.zeros((16,), jnp.float32)
    for k in range(16):
      S = S + plsc.load_gather(
          acc_m, [p * 16 + lane, jnp.full((16,), k, jnp.int32)])
    W = jnp.exp(S)
    for j in range(16):
      e = g * 16 + j
      wv = jnp.broadcast_to(W[j], (16,))
      st[e, pl.ds(52, 16)] = wv
      for k in range(4):
        st[e, pl.ds(k * 16, 16)] = rl[e, pl.ds(k * 16, 16)] * wv

  def compute_half(p):
    rl, rr, st = rows_l[p], rows_r[p], stage[p]

    def group2(h2, _):
      pass1(h2 * 2, 0, rl, rr)
      pass1(h2 * 2 + 1, 1, rl, rr)
      pass2(h2 * 2, 0, rl, st)
      pass2(h2 * 2 + 1, 1, rl, st)
      return 0

    lax.fori_loop(0, GH // 2, group2, 0)

  issue_idx(0, 0)
  issue_idx(1, 1)
  wait_idx(0)
  issue_gather(0)

  def snapshot(p):
    for q in range(B1H // 16):
      dst_scs[p][pl.ds(q * 16, 16)] = dst_vs[p][pl.ds(q * 16, 16)]

  def pipeline(i2, carry):
    wait_idx(1)
    issue_gather(1)
    wait_gather(0)

    @pl.when(i2 >= 1)
    def _():
      wait_scatter(0)

    snapshot(0)

    @pl.when(i2 < NPAIR - 1)
    def _():
      issue_idx(i2 * 2 + 2, 0)

    compute_half(0)
    issue_scatter(0)

    @pl.when(i2 < NPAIR - 1)
    def _():
      wait_idx(0)
      issue_gather(0)

    wait_gather(1)

    @pl.when(i2 >= 1)
    def _():
      wait_scatter(1)

    snapshot(1)

    @pl.when(i2 < NPAIR - 1)
    def _():
      issue_idx(i2 * 2 + 3, 1)

    compute_half(1)
    issue_scatter(1)
    return carry

  lax.fori_loop(0, NPAIR, pipeline, 0)
  wait_scatter(0)
  wait_scatter(1)

  cp1 = pltpu.async_copy(xl_hbm.at[src_t], rows_l0.at[pl.ds(0, 16)], sg0)
  cp2 = pltpu.async_copy(xr_hbm.at[dst_t], rows_r0.at[pl.ds(0, 16)], sg1)
  cp1.wait()
  cp2.wait()
  pass1(0, 0, rows_l0, rows_r0)
  pass2(0, 0, rows_l0, stage0)
  pltpu.sync_copy(stage0.at[pl.ds(0, 16)], numer_s.at[dst_t], add=True)

  plsc.subcore_barrier()

  @pl.when(s < 10)
  def _():
    pltpu.sync_copy(numer_s.at[pl.ds(s * ROWS_PER_CP, ROWS_PER_CP)],
                    numer_out.at[pl.ds(c * N + s * ROWS_PER_CP, ROWS_PER_CP)])


def _l2_edges(tab_hbm, src_hbm, dst_hbm, att2v_hbm, zero4_hbm, acc_out,
              tab_v, src_v, dst_v, rows2, att2_v, acc_s, sem1):
  c = lax.axis_index("c")
  s = lax.axis_index("s")
  wid = c * NS + s

  @pl.when(s < 10)
  def _():
    pltpu.sync_copy(zero4_hbm.at[pl.ds(s * ROWS_PER_CP, ROWS_PER_CP)],
                    acc_s.at[pl.ds(s * ROWS_PER_CP, ROWS_PER_CP)])

  pltpu.sync_copy(tab_hbm, tab_v)
  pltpu.sync_copy(att2v_hbm, att2_v)
  plsc.subcore_barrier()

  lane = jnp.arange(16, dtype=jnp.int32)
  i0 = jnp.zeros((16,), jnp.int32)
  i1 = i0 + 1
  i2 = i0 + 2
  i3 = i0 + 3
  a0 = att2_v[0, :]
  a1 = att2_v[1, :]
  zf = jnp.zeros((16,), jnp.float32)

  def chunk(i, carry):
    base = wid * PER_W + i * B2
    pltpu.sync_copy(src_hbm.at[pl.ds(base, B2)], src_v)
    pltpu.sync_copy(dst_hbm.at[pl.ds(base, B2)], dst_v)

    def group(g, _):
      sv = src_v[pl.ds(g * 16, 16)]
      dv = dst_v[pl.ds(g * 16, 16)]
      l0 = plsc.load_gather(tab_v, [sv, i0])
      l1 = plsc.load_gather(tab_v, [sv, i1])
      r0 = plsc.load_gather(tab_v, [dv, i2])
      r1 = plsc.load_gather(tab_v, [dv, i3])
      t0 = l0 + r0
      t0 = jnp.maximum(t0, t0 * 0.2)
      t1 = l1 + r1
      t1 = jnp.maximum(t1, t1 * 0.2)
      w = jnp.exp(a0 * t0 + a1 * t1)
      eidx = g * 16 + lane
      plsc.store_scatter(rows2, [eidx, i0], w * l0)
      plsc.store_scatter(rows2, [eidx, i1], w * l1)
      plsc.store_scatter(rows2, [eidx, i2], w)
      plsc.store_scatter(rows2, [eidx, i3], zf)
      return 0

    lax.fori_loop(0, G2, group, 0)
    pltpu.sync_copy(rows2, acc_s.at[dst_v], add=True)
    return carry

  lax.fori_loop(0, NCH2, chunk, 0)
  plsc.subcore_barrier()

  @pl.when(s < 10)
  def _():
    pltpu.sync_copy(acc_s.at[pl.ds(s * ROWS_PER_CP, ROWS_PER_CP)],
                    acc_out.at[pl.ds(c * N + s * ROWS_PER_CP, ROWS_PER_CP)])


def _tc_in_body(x_ref, w_ref, b_ref, xl_ref, xr_ref):
  y = jnp.dot(x_ref[...], w_ref[...], preferred_element_type=jnp.float32)
  y = y + b_ref[...]
  xl_ref[...] = y[:, :HID]
  xr_ref[...] = y[:, HID:]


def _tc_mid_body(p0_ref, p1_ref, b1_ref, w2_ref, b2_ref, o_ref):
  p = p0_ref[...] + p1_ref[...]
  h = p[:, :HID] / (p[:, HID:HID + 1] + 1e-16)
  h = h + b1_ref[...]
  h = jnp.where(h > 0, h, jnp.exp(jnp.minimum(h, 0.0)) - 1.0)
  o_ref[...] = (
      jnp.dot(h, w2_ref[...], preferred_element_type=jnp.float32)
      + b2_ref[...]
  )


def _tc_fin_body(a0_ref, a1_ref, b_ref, o_ref):
  a = a0_ref[...] + a1_ref[...]
  o_ref[...] = a[:, :OUT] / (a[:, OUT:OUT + 1] + 1e-16) + b_ref[...]


_ROWBLK = 2000


def _tc_in(x, wcat_t, bcat):
  return pl.pallas_call(
      _tc_in_body,
      grid=(N // _ROWBLK,),
      in_specs=[
          pl.BlockSpec((_ROWBLK, IN), lambda i: (i, 0)),
          pl.BlockSpec((IN, 2 * HID), lambda i: (0, 0)),
          pl.BlockSpec((1, 2 * HID), lambda i: (0, 0)),
      ],
      out_specs=[
          pl.BlockSpec((_ROWBLK, HID), lambda i: (i, 0)),
          pl.BlockSpec((_ROWBLK, HID), lambda i: (i, 0)),
      ],
      out_shape=[
          jax.ShapeDtypeStruct((N, HID), jnp.float32),
          jax.ShapeDtypeStruct((N, HID), jnp.float32),
      ],
  )(x, wcat_t, bcat)


def _tc_mid(p0, p1, b1, w2t, b2):
  return pl.pallas_call(
      _tc_mid_body,
      grid=(N // _ROWBLK,),
      in_specs=[
          pl.BlockSpec((_ROWBLK, 68), lambda i: (i, 0)),
          pl.BlockSpec((_ROWBLK, 68), lambda i: (i, 0)),
          pl.BlockSpec((1, HID), lambda i: (0, 0)),
          pl.BlockSpec((HID, 4), lambda i: (0, 0)),
          pl.BlockSpec((1, 4), lambda i: (0, 0)),
      ],
      out_specs=pl.BlockSpec((_ROWBLK, 4), lambda i: (i, 0)),
      out_shape=jax.ShapeDtypeStruct((N, 4), jnp.float32),
  )(p0, p1, b1, w2t, b2)


def _tc_fin(a0, a1, b2):
  return pl.pallas_call(
      _tc_fin_body,
      grid=(N // _ROWBLK,),
      in_specs=[
          pl.BlockSpec((_ROWBLK, 4), lambda i: (i, 0)),
          pl.BlockSpec((_ROWBLK, 4), lambda i: (i, 0)),
          pl.BlockSpec((1, OUT), lambda i: (0, 0)),
      ],
      out_specs=pl.BlockSpec((_ROWBLK, OUT), lambda i: (i, 0)),
      out_shape=jax.ShapeDtypeStruct((N, OUT), jnp.float32),
  )(a0, a1, b2)


def kernel(x, edge_index, batch, Wl1, bl1, Wr1, br1, att1, bias1,
           Wl2, bl2, Wr2, br2, att2, bias2):
  del batch
  src = edge_index[0]
  dst = edge_index[1]

  wcat_t = jnp.concatenate([Wl1, Wr1], axis=0).T
  bcat = jnp.concatenate([bl1, br1], axis=0)[None, :]
  xl, xr = _tc_in(x, wcat_t, bcat)

  attv = att1.reshape(4, 16)
  zero80 = jnp.zeros((N, 68), jnp.float32)

  srcr = src.reshape(NW, PER_W)
  dstr = dst.reshape(NW, PER_W)
  srct = srcr[:, L1_MAIN:]
  dstt = dstr[:, L1_MAIN:]

  mesh = plsc.VectorSubcoreMesh(core_axis_name="c", subcore_axis_name="s",
                                num_cores=NC, num_subcores=NS)
  sc_params = pltpu.CompilerParams(needs_layout_passes=False,
                                   use_tc_tiling_on_sc=False)
  l1 = pl.kernel(
      _l1_edges,
      compiler_params=sc_params,
      out_type=jax.ShapeDtypeStruct((NC * N, 68), jnp.float32),
      mesh=mesh,
      scratch_types=[
          pltpu.VMEM((B1H,), jnp.int32),
          pltpu.VMEM((B1H,), jnp.int32),
          pltpu.VMEM((B1H,), jnp.int32),
          pltpu.VMEM((B1H,), jnp.int32),
          pltpu.VMEM((B1H,), jnp.int32),
          pltpu.VMEM((B1H,), jnp.int32),
          pltpu.VMEM((L1_TAIL,), jnp.int32),
          pltpu.VMEM((L1_TAIL,), jnp.int32),
          pltpu.VMEM((B1H, HID), jnp.float32),
          pltpu.VMEM((B1H, HID), jnp.float32),
          pltpu.VMEM((B1H, HID), jnp.float32),
          pltpu.VMEM((B1H, HID), jnp.float32),
          pltpu.VMEM((B1H, 68), jnp.float32),
          pltpu.VMEM((B1H, 68), jnp.float32),
          pltpu.VMEM((4, 16), jnp.float32),
          pltpu.VMEM((32, 17), jnp.float32),
          pltpu.MemorySpace.VMEM_SHARED((N, 68), jnp.float32),
          pltpu.SemaphoreType.DMA,
          pltpu.SemaphoreType.DMA,
          pltpu.SemaphoreType.DMA,
          pltpu.SemaphoreType.DMA,
          pltpu.SemaphoreType.DMA,
          pltpu.SemaphoreType.DMA,
      ],
  )
  numer = l1(xl, xr, src, dst, srct, dstt, attv, zero80)

  w2t = jnp.concatenate([Wl2, Wr2], axis=0).T
  b2 = jnp.concatenate([bl2, br2], axis=0)[None, :]
  tab2 = _tc_mid(numer[:N], numer[N:], bias1[None, :], w2t, b2)

  att2v = jnp.broadcast_to(att2.reshape(2, 1), (2, 16))
  zero4 = jnp.zeros((N, 4), jnp.float32)

  l2 = pl.kernel(
      _l2_edges,
      compiler_params=sc_params,
      out_type=jax.ShapeDtypeStruct((NC * N, 4), jnp.float32),
      mesh=mesh,
      scratch_types=[
          pltpu.VMEM((N, 4), jnp.float32),
          pltpu.VMEM((B2,), jnp.int32),
          pltpu.VMEM((B2,), jnp.int32),
          pltpu.VMEM((B2, 4), jnp.float32),
          pltpu.VMEM((2, 16), jnp.float32),
          pltpu.MemorySpace.VMEM_SHARED((N, 4), jnp.float32),
          pltpu.SemaphoreType.DMA,
      ],
  )
  acc = l2(tab2, src, dst, att2v, zero4)

  return _tc_fin(acc[:N], acc[N:], bias2[None, :])
